# baseline (speedup 1.0000x reference)
import functools

import jax
import jax.numpy as jnp
from jax import lax
from jax.experimental import pallas as pl
from jax.experimental.pallas import tpu as pltpu
from jax.experimental.pallas import tpu_sc as plsc

NUM_RELATIONS = 1000
D = 64
B = 16384
NC = 2
NS = 16
L = 16
NW = NC * NS
BPW = B // NW
NCHUNK = 4
CB = BPW // NCHUNK
NBUF = 2

_mesh = plsc.VectorSubcoreMesh(core_axis_name="c", subcore_axis_name="s")


@functools.partial(
    pl.kernel,
    mesh=_mesh,
    out_type=jax.ShapeDtypeStruct((B,), jnp.float32),
    compiler_params=pltpu.CompilerParams(needs_layout_passes=False),
    scratch_types=[
        pltpu.VMEM((BPW,), jnp.int32),
        pltpu.VMEM((D, NUM_RELATIONS), jnp.float32),
        pltpu.VMEM((NBUF, D, CB), jnp.float32),
        pltpu.VMEM((NBUF, D, CB), jnp.float32),
        pltpu.VMEM((BPW,), jnp.float32),
        pltpu.SemaphoreType.DMA,
    ] + [pltpu.SemaphoreType.DMA] * NCHUNK,
)
def _distmult_sc(head_hbm, tail_hbm, idx_hbm, table_hbm, out_hbm,
                 idx_v, table_v, head_v, tail_v, out_v, sem0, *sems):
    wid = lax.axis_index("s") * NC + lax.axis_index("c")
    base = wid * BPW

    tbl_cp = pltpu.async_copy(table_hbm, table_v, sem0)
    idx_cp = pltpu.async_copy(idx_hbm.at[pl.ds(base, BPW)], idx_v, sem0)

    def issue(c):
        b0 = base + c * CB
        slot = c % NBUF
        return (
            pltpu.async_copy(head_hbm.at[:, pl.ds(b0, CB)],
                             head_v.at[slot], sems[c]),
            pltpu.async_copy(tail_hbm.at[:, pl.ds(b0, CB)],
                             tail_v.at[slot], sems[c]),
        )

    copies = {0: issue(0), 1: issue(1)}
    with jax.named_scope("tblwait"):
        tbl_cp.wait()
        idx_cp.wait()

    for c in range(NCHUNK):
        slot = c % NBUF
        with jax.named_scope(f"wait{c}"):
            for cp in copies[c]:
                cp.wait()

        scope = jax.named_scope(f"comp{c}")
        scope.__enter__()

        @plsc.parallel_loop(0, CB // L, unroll=2)
        def _j_body(j, slot=slot, c=c):
            b0 = c * CB + j * L
            idxv = idx_v[pl.ds(b0, L)]
            accs = [jnp.zeros((L,), jnp.float32) for _ in range(4)]
            for d in range(D):
                rv = plsc.load_gather(
                    table_v, [jnp.full((L,), d, jnp.int32), idxv])
                hv = head_v[slot, d, pl.ds(j * L, L)]
                tv = tail_v[slot, d, pl.ds(j * L, L)]
                accs[d % 4] = accs[d % 4] + hv * rv * tv
            out_v[pl.ds(b0, L)] = (accs[0] + accs[1]) + (accs[2] + accs[3])
        scope.__exit__(None, None, None)
        if c + NBUF < NCHUNK:
            copies[c + NBUF] = issue(c + NBUF)

    pltpu.sync_copy(out_v, out_hbm.at[pl.ds(base, BPW)])


def kernel(head_emb, tail_emb, rel_idx, relation_embeddings):
    idx = rel_idx.astype(jnp.int32)
    return _distmult_sc(head_emb.T, tail_emb.T, idx, relation_embeddings.T)


# speedup vs baseline: 1.2374x; 1.2374x over previous
import functools

import jax
import jax.numpy as jnp
from jax import lax
from jax.experimental import pallas as pl
from jax.experimental.pallas import tpu as pltpu
from jax.experimental.pallas import tpu_sc as plsc

NUM_RELATIONS = 1000
D = 64
DP = D // 2
B = 16384
NC = 2
NS = 16
L = 16
NW = NC * NS
BPW = B // NW
NCHUNK = 4
CB = BPW // NCHUNK
NBUF = 2

_mesh = plsc.VectorSubcoreMesh(core_axis_name="c", subcore_axis_name="s")


@functools.partial(
    pl.kernel,
    mesh=_mesh,
    out_type=jax.ShapeDtypeStruct((B,), jnp.float32),
    compiler_params=pltpu.CompilerParams(needs_layout_passes=False),
    scratch_types=[
        pltpu.VMEM((BPW,), jnp.int32),
        pltpu.VMEM((DP, NUM_RELATIONS), jnp.int32),
        pltpu.VMEM((NBUF, D, CB), jnp.float32),
        pltpu.VMEM((NBUF, D, CB), jnp.float32),
        pltpu.VMEM((BPW,), jnp.float32),
        pltpu.SemaphoreType.DMA,
    ] + [pltpu.SemaphoreType.DMA] * NCHUNK,
)
def _distmult_sc(head_hbm, tail_hbm, idx_hbm, table_hbm, out_hbm,
                 idx_v, table_v, head_v, tail_v, out_v, sem0, *sems):
    wid = lax.axis_index("s") * NC + lax.axis_index("c")
    base = wid * BPW

    tbl_cp = pltpu.async_copy(table_hbm, table_v, sem0)
    idx_cp = pltpu.async_copy(idx_hbm.at[pl.ds(base, BPW)], idx_v, sem0)

    def issue(c):
        b0 = base + c * CB
        slot = c % NBUF
        return (
            pltpu.async_copy(head_hbm.at[:, pl.ds(b0, CB)],
                             head_v.at[slot], sems[c]),
            pltpu.async_copy(tail_hbm.at[:, pl.ds(b0, CB)],
                             tail_v.at[slot], sems[c]),
        )

    copies = {0: issue(0), 1: issue(1)}
    with jax.named_scope("tblwait"):
        tbl_cp.wait()
        idx_cp.wait()

    himask = jnp.full((L,), jnp.int32(-65536))

    for c in range(NCHUNK):
        slot = c % NBUF
        with jax.named_scope(f"wait{c}"):
            for cp in copies[c]:
                cp.wait()

        scope = jax.named_scope(f"comp{c}")
        scope.__enter__()

        @plsc.parallel_loop(0, CB // L, unroll=2)
        def _j_body(j, slot=slot, c=c):
            b0 = c * CB + j * L
            idxv = idx_v[pl.ds(b0, L)]
            accs = [jnp.zeros((L,), jnp.float32) for _ in range(4)]
            for dp in range(DP):
                pv = plsc.load_gather(
                    table_v, [jnp.full((L,), dp, jnp.int32), idxv])
                r_lo = plsc.bitcast(lax.shift_left(pv, 16), jnp.float32)
                r_hi = plsc.bitcast(lax.bitwise_and(pv, himask), jnp.float32)
                d0 = 2 * dp
                h0 = head_v[slot, d0, pl.ds(j * L, L)]
                t0 = tail_v[slot, d0, pl.ds(j * L, L)]
                h1 = head_v[slot, d0 + 1, pl.ds(j * L, L)]
                t1 = tail_v[slot, d0 + 1, pl.ds(j * L, L)]
                accs[dp % 4] = accs[dp % 4] + (h0 * r_lo * t0 + h1 * r_hi * t1)
            out_v[pl.ds(b0, L)] = (accs[0] + accs[1]) + (accs[2] + accs[3])
        scope.__exit__(None, None, None)
        if c + NBUF < NCHUNK:
            copies[c + NBUF] = issue(c + NBUF)

    pltpu.sync_copy(out_v, out_hbm.at[pl.ds(base, BPW)])


def kernel(head_emb, tail_emb, rel_idx, relation_embeddings):
    idx = rel_idx.astype(jnp.int32)
    tbl = relation_embeddings.T.astype(jnp.bfloat16)
    pairs = tbl.reshape(DP, 2, NUM_RELATIONS).transpose(0, 2, 1)
    packed = lax.bitcast_convert_type(pairs, jnp.int32)
    return _distmult_sc(head_emb.T, tail_emb.T, idx, packed)
